# baseline (speedup 1.0000x reference)
import jax
import jax.numpy as jnp
from jax import lax
from jax.experimental import pallas as pl
from jax.experimental.pallas import tpu as pltpu
from jax.experimental.pallas import tpu_sc as plsc

VOCAB = 100000
HIDDEN = 128
VIS_DIM = 2048
BATCH = 1024
L_TEXT = 200
L_IMG = 50
L_OUT = L_IMG + L_TEXT
EPS = 1e-12

NHALF = 4
HB = BATCH // NHALF

NUM_WORKERS = 32
CHUNK = 64
TOK_H = HB * L_TEXT
TPW = TOK_H // NUM_WORKERS
NCHUNK = TPW // CHUNK

BB = 32
IMG_POS_W = 64
TXT_POS_W = 256


def _sc_gather_body(tbl_hbm, idx_hbm, out_hbm, idx_v, rows_a, rows_b,
                    sem_a, sem_b):
    wid = lax.axis_index("s") * 2 + lax.axis_index("c")
    base = wid * TPW
    pltpu.sync_copy(idx_hbm.at[wid], idx_v)

    NEVEN = NCHUNK - (NCHUNK % 2)

    @pl.loop(0, NEVEN, step=2)
    def _(j):
        cp_a = pltpu.async_copy(tbl_hbm.at[idx_v.at[j]], rows_a, sem_a)
        cp_b = pltpu.async_copy(tbl_hbm.at[idx_v.at[j + 1]], rows_b, sem_b)
        cp_a.wait()
        pltpu.sync_copy(rows_a, out_hbm.at[pl.ds(base + j * CHUNK, CHUNK)])
        cp_b.wait()
        pltpu.sync_copy(rows_b, out_hbm.at[pl.ds(base + (j + 1) * CHUNK, CHUNK)])

    if NCHUNK % 2:
        j = NCHUNK - 1
        pltpu.async_copy(tbl_hbm.at[idx_v.at[j]], rows_a, sem_a).wait()
        pltpu.sync_copy(rows_a, out_hbm.at[pl.ds(base + j * CHUNK, CHUNK)])


def _sc_gather(word_emb, flat_ids):
    idx = flat_ids.reshape(NUM_WORKERS, NCHUNK, CHUNK)
    mesh = plsc.VectorSubcoreMesh(core_axis_name="c", subcore_axis_name="s")
    run = pl.kernel(
        _sc_gather_body,
        out_type=jax.ShapeDtypeStruct((TOK_H, HIDDEN), jnp.float32),
        mesh=mesh,
        scratch_types=[
            pltpu.VMEM((NCHUNK, CHUNK), jnp.int32),
            pltpu.VMEM((CHUNK, HIDDEN), jnp.float32),
            pltpu.VMEM((CHUNK, HIDDEN), jnp.float32),
            pltpu.SemaphoreType.DMA,
            pltpu.SemaphoreType.DMA,
        ],
    )
    return run(word_emb, idx)


def _ln(x, g, b):
    mu = jnp.mean(x, axis=-1, keepdims=True)
    xc = x - mu
    var = jnp.mean(xc * xc, axis=-1, keepdims=True)
    return xc * lax.rsqrt(var + EPS) * g + b


def _onehot_tm(ids_ref, seg_ref, length, width):
    ids_t = jnp.transpose(ids_ref[...])
    seg_t = jnp.transpose(seg_ref[...]).astype(jnp.float32)
    iota = lax.broadcasted_iota(jnp.int32, (length, BB, width), 2)
    oh = (ids_t[:, :, None] == iota).astype(jnp.float32)
    oh = oh + seg_t[:, :, None] * (iota == width - 1).astype(jnp.float32)
    return oh.reshape(length * BB, width)


def _tc_body(gat_ref, feat_ref, w_ref, bias_ref, eg_ref, eb_ref,
             ipt_ref, tpt_ref, ipos_ref, tpos_ref, iseg_ref, tseg_ref,
             tt0_ref, ig_ref, ib_ref, tg_ref, tb_ref, out_ref):
    feat = feat_ref[...].reshape(L_IMG * BB, VIS_DIM)
    proj = jnp.dot(feat, w_ref[...], preferred_element_type=jnp.float32)
    proj = proj + bias_ref[0]
    x = _ln(proj, eg_ref[0], eb_ref[0])
    ioh = _onehot_tm(ipos_ref, iseg_ref, L_IMG, IMG_POS_W)
    x = x + jnp.dot(ioh, ipt_ref[...], preferred_element_type=jnp.float32)
    x = x + tt0_ref[0]
    x = _ln(x, ig_ref[0], ib_ref[0])
    out_ref[:L_IMG] = x.reshape(L_IMG, BB, HIDDEN)

    t = gat_ref[...].reshape(L_TEXT * BB, HIDDEN)
    toh = _onehot_tm(tpos_ref, tseg_ref, L_TEXT, TXT_POS_W)
    t = t + jnp.dot(toh, tpt_ref[...], preferred_element_type=jnp.float32)
    t = t + tt0_ref[0]
    t = _ln(t, tg_ref[0], tb_ref[0])
    out_ref[L_IMG:] = t.reshape(L_TEXT, BB, HIDDEN)


def _tc_alias_body(*refs):
    _tc_body(*refs[:17], refs[-1])


def _tc_fused_half(half, gathered_h, feat_t, image_W, image_b,
                   image_emb_ln_g, image_emb_ln_b, img_tab, txt_tab,
                   image_position_ids, text_position_ids,
                   image_segment_ids, text_segment_ids,
                   tt0, image_ln_g, image_ln_b, text_ln_g, text_ln_b,
                   out_prev=None):
    row = lambda b: (0, 0)
    boff = half * (HB // BB)
    in_specs = [
        pl.BlockSpec((L_TEXT, BB, HIDDEN), lambda b: (0, b, 0)),
        pl.BlockSpec((L_IMG, BB, VIS_DIM), lambda b: (0, boff + b, 0)),
        pl.BlockSpec((VIS_DIM, HIDDEN), row),
        pl.BlockSpec((1, HIDDEN), row),
        pl.BlockSpec((1, HIDDEN), row),
        pl.BlockSpec((1, HIDDEN), row),
        pl.BlockSpec((IMG_POS_W, HIDDEN), row),
        pl.BlockSpec((TXT_POS_W, HIDDEN), row),
        pl.BlockSpec((BB, L_IMG), lambda b: (boff + b, 0)),
        pl.BlockSpec((BB, L_TEXT), lambda b: (boff + b, 0)),
        pl.BlockSpec((BB, L_IMG), lambda b: (boff + b, 0)),
        pl.BlockSpec((BB, L_TEXT), lambda b: (boff + b, 0)),
        pl.BlockSpec((1, HIDDEN), row),
        pl.BlockSpec((1, HIDDEN), row),
        pl.BlockSpec((1, HIDDEN), row),
        pl.BlockSpec((1, HIDDEN), row),
        pl.BlockSpec((1, HIDDEN), row),
    ]
    args = [gathered_h, feat_t, image_W, image_b,
            image_emb_ln_g, image_emb_ln_b, img_tab, txt_tab,
            image_position_ids, text_position_ids,
            image_segment_ids, text_segment_ids,
            tt0, image_ln_g, image_ln_b, text_ln_g, text_ln_b]
    body = _tc_body
    aliases = {}
    if out_prev is not None:
        in_specs.append(pl.BlockSpec(memory_space=pltpu.MemorySpace.HBM))
        args.append(out_prev)
        aliases = {17: 0}
        body = _tc_alias_body
    return pl.pallas_call(
        body,
        grid=(HB // BB,),
        in_specs=in_specs,
        out_specs=pl.BlockSpec((L_OUT, BB, HIDDEN), lambda b: (0, boff + b, 0)),
        out_shape=jax.ShapeDtypeStruct((L_OUT, BATCH, HIDDEN), jnp.float32),
        input_output_aliases=aliases,
    )(*args)


def kernel(text_input_ids, image_input_feat, text_position_ids,
           image_position_ids, text_segment_ids, image_segment_ids,
           word_emb, image_W, image_b, image_emb_ln_g, image_emb_ln_b,
           text_pos_emb, image_pos_emb, token_type_emb,
           image_ln_g, image_ln_b, text_ln_g, text_ln_b):
    ids_t = jnp.transpose(text_input_ids)
    gathered = [
        _sc_gather(word_emb, ids_t[:, h * HB:(h + 1) * HB].reshape(-1))
        .reshape(L_TEXT, HB, HIDDEN)
        for h in range(NHALF)
    ]

    feat_t = jnp.transpose(image_input_feat, (1, 0, 2))

    tt0 = token_type_emb[0:1]
    ttd_row = token_type_emb[1] - token_type_emb[0]
    img_tab = image_pos_emb[:IMG_POS_W].at[IMG_POS_W - 1].set(ttd_row)
    txt_tab = text_pos_emb[:TXT_POS_W].at[TXT_POS_W - 1].set(ttd_row)

    common = (feat_t, image_W, image_b.reshape(1, HIDDEN),
              image_emb_ln_g.reshape(1, HIDDEN),
              image_emb_ln_b.reshape(1, HIDDEN),
              img_tab, txt_tab,
              image_position_ids, text_position_ids,
              image_segment_ids, text_segment_ids,
              tt0,
              image_ln_g.reshape(1, HIDDEN), image_ln_b.reshape(1, HIDDEN),
              text_ln_g.reshape(1, HIDDEN), text_ln_b.reshape(1, HIDDEN))

    out_t = _tc_fused_half(0, gathered[0], *common)
    for h in range(1, NHALF):
        out_t = _tc_fused_half(h, gathered[h], *common, out_prev=out_t)
    return jnp.transpose(out_t, (1, 0, 2))


# speedup vs baseline: 1.0321x; 1.0321x over previous
import jax
import jax.numpy as jnp
from jax import lax
from jax.experimental import pallas as pl
from jax.experimental.pallas import tpu as pltpu
from jax.experimental.pallas import tpu_sc as plsc

VOCAB = 100000
HIDDEN = 128
VIS_DIM = 2048
BATCH = 1024
L_TEXT = 200
L_IMG = 50
L_OUT = L_IMG + L_TEXT
EPS = 1e-12

NHALF = 2
HB = BATCH // NHALF

NUM_WORKERS = 32
CHUNK = 128
TOK_H = HB * L_TEXT
TPW = TOK_H // NUM_WORKERS
NCHUNK = TPW // CHUNK

BB = 32
IMG_POS_W = 64
TXT_POS_W = 256


def _sc_gather_body(tbl_hbm, idx_hbm, out_hbm, idx_v, rows_a, rows_b,
                    sem_a, sem_b):
    wid = lax.axis_index("s") * 2 + lax.axis_index("c")
    base = wid * TPW
    pltpu.sync_copy(idx_hbm.at[wid], idx_v)

    NEVEN = NCHUNK - (NCHUNK % 2)

    @pl.loop(0, NEVEN, step=2)
    def _(j):
        cp_a = pltpu.async_copy(tbl_hbm.at[idx_v.at[j]], rows_a, sem_a)
        cp_b = pltpu.async_copy(tbl_hbm.at[idx_v.at[j + 1]], rows_b, sem_b)
        cp_a.wait()
        pltpu.sync_copy(rows_a, out_hbm.at[pl.ds(base + j * CHUNK, CHUNK)])
        cp_b.wait()
        pltpu.sync_copy(rows_b, out_hbm.at[pl.ds(base + (j + 1) * CHUNK, CHUNK)])

    if NCHUNK % 2:
        j = NCHUNK - 1
        pltpu.async_copy(tbl_hbm.at[idx_v.at[j]], rows_a, sem_a).wait()
        pltpu.sync_copy(rows_a, out_hbm.at[pl.ds(base + j * CHUNK, CHUNK)])


def _sc_gather(word_emb, flat_ids):
    idx = flat_ids.reshape(NUM_WORKERS, NCHUNK, CHUNK)
    mesh = plsc.VectorSubcoreMesh(core_axis_name="c", subcore_axis_name="s")
    run = pl.kernel(
        _sc_gather_body,
        out_type=jax.ShapeDtypeStruct((TOK_H, HIDDEN), jnp.float32),
        mesh=mesh,
        scratch_types=[
            pltpu.VMEM((NCHUNK, CHUNK), jnp.int32),
            pltpu.VMEM((CHUNK, HIDDEN), jnp.float32),
            pltpu.VMEM((CHUNK, HIDDEN), jnp.float32),
            pltpu.SemaphoreType.DMA,
            pltpu.SemaphoreType.DMA,
        ],
    )
    return run(word_emb, idx)


def _ln(x, g, b):
    mu = jnp.mean(x, axis=-1, keepdims=True)
    xc = x - mu
    var = jnp.mean(xc * xc, axis=-1, keepdims=True)
    return xc * lax.rsqrt(var + EPS) * g + b


def _onehot_tm(ids_ref, seg_ref, length, width):
    ids_t = jnp.transpose(ids_ref[...])
    seg_t = jnp.transpose(seg_ref[...]).astype(jnp.float32)
    iota = lax.broadcasted_iota(jnp.int32, (length, BB, width), 2)
    oh = (ids_t[:, :, None] == iota).astype(jnp.float32)
    oh = oh + seg_t[:, :, None] * (iota == width - 1).astype(jnp.float32)
    return oh.reshape(length * BB, width)


def _tc_body(gat_ref, feat_ref, w_ref, bias_ref, eg_ref, eb_ref,
             ipt_ref, tpt_ref, ipos_ref, tpos_ref, iseg_ref, tseg_ref,
             tt0_ref, ig_ref, ib_ref, tg_ref, tb_ref, out_ref):
    feat = feat_ref[...].reshape(L_IMG * BB, VIS_DIM)
    proj = jnp.dot(feat, w_ref[...], preferred_element_type=jnp.float32)
    proj = proj + bias_ref[0]
    x = _ln(proj, eg_ref[0], eb_ref[0])
    ioh = _onehot_tm(ipos_ref, iseg_ref, L_IMG, IMG_POS_W)
    x = x + jnp.dot(ioh, ipt_ref[...], preferred_element_type=jnp.float32)
    x = x + tt0_ref[0]
    x = _ln(x, ig_ref[0], ib_ref[0])
    out_ref[:L_IMG] = x.reshape(L_IMG, BB, HIDDEN)

    t = gat_ref[...].reshape(L_TEXT * BB, HIDDEN)
    toh = _onehot_tm(tpos_ref, tseg_ref, L_TEXT, TXT_POS_W)
    t = t + jnp.dot(toh, tpt_ref[...], preferred_element_type=jnp.float32)
    t = t + tt0_ref[0]
    t = _ln(t, tg_ref[0], tb_ref[0])
    out_ref[L_IMG:] = t.reshape(L_TEXT, BB, HIDDEN)


def _tc_alias_body(*refs):
    _tc_body(*refs[:17], refs[-1])


def _tc_fused_half(half, gathered_h, feat_t, image_W, image_b,
                   image_emb_ln_g, image_emb_ln_b, img_tab, txt_tab,
                   image_position_ids, text_position_ids,
                   image_segment_ids, text_segment_ids,
                   tt0, image_ln_g, image_ln_b, text_ln_g, text_ln_b,
                   out_prev=None):
    row = lambda b: (0, 0)
    boff = half * (HB // BB)
    in_specs = [
        pl.BlockSpec((L_TEXT, BB, HIDDEN), lambda b: (0, b, 0)),
        pl.BlockSpec((L_IMG, BB, VIS_DIM), lambda b: (0, boff + b, 0)),
        pl.BlockSpec((VIS_DIM, HIDDEN), row),
        pl.BlockSpec((1, HIDDEN), row),
        pl.BlockSpec((1, HIDDEN), row),
        pl.BlockSpec((1, HIDDEN), row),
        pl.BlockSpec((IMG_POS_W, HIDDEN), row),
        pl.BlockSpec((TXT_POS_W, HIDDEN), row),
        pl.BlockSpec((BB, L_IMG), lambda b: (boff + b, 0)),
        pl.BlockSpec((BB, L_TEXT), lambda b: (boff + b, 0)),
        pl.BlockSpec((BB, L_IMG), lambda b: (boff + b, 0)),
        pl.BlockSpec((BB, L_TEXT), lambda b: (boff + b, 0)),
        pl.BlockSpec((1, HIDDEN), row),
        pl.BlockSpec((1, HIDDEN), row),
        pl.BlockSpec((1, HIDDEN), row),
        pl.BlockSpec((1, HIDDEN), row),
        pl.BlockSpec((1, HIDDEN), row),
    ]
    args = [gathered_h, feat_t, image_W, image_b,
            image_emb_ln_g, image_emb_ln_b, img_tab, txt_tab,
            image_position_ids, text_position_ids,
            image_segment_ids, text_segment_ids,
            tt0, image_ln_g, image_ln_b, text_ln_g, text_ln_b]
    body = _tc_body
    aliases = {}
    if out_prev is not None:
        in_specs.append(pl.BlockSpec(memory_space=pltpu.MemorySpace.HBM))
        args.append(out_prev)
        aliases = {17: 0}
        body = _tc_alias_body
    return pl.pallas_call(
        body,
        grid=(HB // BB,),
        in_specs=in_specs,
        out_specs=pl.BlockSpec((L_OUT, BB, HIDDEN), lambda b: (0, boff + b, 0)),
        out_shape=jax.ShapeDtypeStruct((L_OUT, BATCH, HIDDEN), jnp.float32),
        input_output_aliases=aliases,
    )(*args)


def kernel(text_input_ids, image_input_feat, text_position_ids,
           image_position_ids, text_segment_ids, image_segment_ids,
           word_emb, image_W, image_b, image_emb_ln_g, image_emb_ln_b,
           text_pos_emb, image_pos_emb, token_type_emb,
           image_ln_g, image_ln_b, text_ln_g, text_ln_b):
    ids_t = jnp.transpose(text_input_ids)
    gathered = [
        _sc_gather(word_emb, ids_t[:, h * HB:(h + 1) * HB].reshape(-1))
        .reshape(L_TEXT, HB, HIDDEN)
        for h in range(NHALF)
    ]

    feat_t = jnp.transpose(image_input_feat, (1, 0, 2))

    tt0 = token_type_emb[0:1]
    ttd_row = token_type_emb[1] - token_type_emb[0]
    img_tab = image_pos_emb[:IMG_POS_W].at[IMG_POS_W - 1].set(ttd_row)
    txt_tab = text_pos_emb[:TXT_POS_W].at[TXT_POS_W - 1].set(ttd_row)

    common = (feat_t, image_W, image_b.reshape(1, HIDDEN),
              image_emb_ln_g.reshape(1, HIDDEN),
              image_emb_ln_b.reshape(1, HIDDEN),
              img_tab, txt_tab,
              image_position_ids, text_position_ids,
              image_segment_ids, text_segment_ids,
              tt0,
              image_ln_g.reshape(1, HIDDEN), image_ln_b.reshape(1, HIDDEN),
              text_ln_g.reshape(1, HIDDEN), text_ln_b.reshape(1, HIDDEN))

    out_t = _tc_fused_half(0, gathered[0], *common)
    for h in range(1, NHALF):
        out_t = _tc_fused_half(h, gathered[h], *common, out_prev=out_t)
    return jnp.transpose(out_t, (1, 0, 2))
